# initial kernel scaffold (unmeasured)
import jax
import jax.numpy as jnp
from jax import lax
from jax.experimental import pallas as pl
from jax.experimental.pallas import tpu as pltpu

N_DEV = 8


def kernel(x, w_mat):
    m, k_per = x.shape
    _, n = w_mat.shape
    m_per = m // N_DEV

    def body(x_ref, w_ref, out_ref, send_buf, recv_buf, send_sems, recv_sems):
        d = lax.axis_index("i")
        left = lax.rem(d + N_DEV - 1, N_DEV)
        right = lax.rem(d + 1, N_DEV)

        barrier_sem = pltpu.get_barrier_semaphore()
        for nbr in (left, right):
            pl.semaphore_signal(
                barrier_sem, inc=1,
                device_id=(nbr,), device_id_type=pl.DeviceIdType.MESH,
            )
        pl.semaphore_wait(barrier_sem, 2)

        def partial_chunk(c):
            return jnp.dot(
                x_ref[pl.ds(c * m_per, m_per), :],
                w_ref[...],
                preferred_element_type=jnp.float32,
            )

        for s in range(N_DEV - 1):
            c = lax.rem(d + 2 * N_DEV - 1 - s, N_DEV)
            val = partial_chunk(c)
            if s > 0:
                val = val + recv_buf[s - 1].astype(jnp.float32)
            send_buf[s % 2] = val.astype(jnp.bfloat16)
            rdma = pltpu.make_async_remote_copy(
                src_ref=send_buf.at[s % 2],
                dst_ref=recv_buf.at[s],
                send_sem=send_sems.at[s % 2],
                recv_sem=recv_sems.at[s],
                device_id=(right,),
                device_id_type=pl.DeviceIdType.MESH,
            )
            rdma.start()
            rdma.wait()

        val = partial_chunk(d) + recv_buf[N_DEV - 2].astype(jnp.float32)
        out_ref[...] = jnp.maximum(val, 0.0)

    return pl.pallas_call(
        body,
        out_shape=jax.ShapeDtypeStruct((m_per, n), jnp.float32),
        in_specs=[
            pl.BlockSpec(memory_space=pltpu.VMEM),
            pl.BlockSpec(memory_space=pltpu.VMEM),
        ],
        out_specs=pl.BlockSpec(memory_space=pltpu.VMEM),
        scratch_shapes=[
            pltpu.VMEM((2, m_per, n), jnp.bfloat16),
            pltpu.VMEM((N_DEV - 1, m_per, n), jnp.bfloat16),
            pltpu.SemaphoreType.DMA((2,)),
            pltpu.SemaphoreType.DMA((N_DEV - 1,)),
        ],
        compiler_params=pltpu.CompilerParams(
            collective_id=0,
            vmem_limit_bytes=192 * 1024 * 1024,
        ),
    )(x, w_mat)


# baseline (device time: 400430 ns/iter reference)
import jax
import jax.numpy as jnp
from jax import lax
from jax.experimental import pallas as pl
from jax.experimental.pallas import tpu as pltpu

N_DEV = 8


def kernel(x, w_mat):
    x = x.astype(jnp.bfloat16)
    w_mat = w_mat.astype(jnp.bfloat16)
    m, k_per = x.shape
    n = w_mat.shape[1]
    m_per = m // N_DEV
    nh = n // 2
    nq = n // 4

    def body(x_hbm, w_ref, out_hbm, xt, send_buf, recv_buf, out_stage,
             xt_sems, send_sems, recv_sems, out_sems, credit_r, credit_l):
        d = lax.axis_index("i")
        left = lax.rem(d + N_DEV - 1, N_DEV)
        right = lax.rem(d + 1, N_DEV)

        dst = (right, left)
        upstream = (left, right)
        credits = (credit_r, credit_l)

        def chunk(dirn, s):
            return lax.rem(d + 7 - s, N_DEV) if dirn == 0 else \
                   lax.rem(d + 1 + s, N_DEV)

        def xt_copy(dirn, s):
            return pltpu.make_async_copy(
                x_hbm.at[pl.ds(chunk(dirn, s) * m_per, m_per), :],
                xt.at[dirn, s % 2],
                xt_sems.at[dirn, s % 2],
            )

        def rdma(dirn, s):
            return pltpu.make_async_remote_copy(
                src_ref=send_buf.at[dirn, s % 2],
                dst_ref=recv_buf.at[dirn, s % 2],
                send_sem=send_sems.at[dirn, s % 2],
                recv_sem=recv_sems.at[dirn, s % 2],
                device_id=(dst[dirn],),
                device_id_type=pl.DeviceIdType.MESH,
            )

        for s in (0, 1):
            for dirn in (0, 1):
                xt_copy(dirn, s).start()

        barrier_sem = pltpu.get_barrier_semaphore()
        for nbr in (left, right):
            pl.semaphore_signal(
                barrier_sem, inc=1,
                device_id=(nbr,), device_id_type=pl.DeviceIdType.MESH,
            )
        pl.semaphore_wait(barrier_sem, 2)

        for s in range(7):
            for dirn in (0, 1):
                if s >= 2:
                    rdma(dirn, s - 2).wait_send()
                xt_copy(dirn, s).wait()
                send_buf[dirn, s % 2] = jnp.dot(
                    xt[dirn, s % 2],
                    w_ref[:, dirn * nh:(dirn + 1) * nh],
                    preferred_element_type=jnp.float32,
                ).astype(jnp.bfloat16)
            if s + 2 <= 7:
                for dirn in (0, 1):
                    xt_copy(dirn, s + 2).start()
            for dirn in (0, 1):
                if s > 0:
                    rdma(dirn, s - 1).wait_recv()
                    send_buf[dirn, s % 2] = (
                        send_buf[dirn, s % 2] + recv_buf[dirn, (s - 1) % 2]
                    )
                    if s - 1 <= 4:
                        pl.semaphore_signal(
                            credits[dirn], inc=1,
                            device_id=(upstream[dirn],),
                            device_id_type=pl.DeviceIdType.MESH,
                        )
                if s >= 2:
                    pl.semaphore_wait(credits[dirn], 1)
                rdma(dirn, s).start()

        for dirn in (0, 1):
            xt_copy(dirn, 7).wait()
            rdma(dirn, 6).wait_recv()
        for q in range(4):
            dirn = q // 2
            col = (q % 2) * nq
            if q >= 2:
                pltpu.make_async_copy(
                    out_stage.at[q % 2], out_hbm.at[:, pl.ds((q - 2) * nq, nq)],
                    out_sems.at[q % 2],
                ).wait()
            val = jnp.dot(
                xt[dirn, 1],
                w_ref[:, dirn * nh + col:dirn * nh + col + nq],
                preferred_element_type=jnp.float32,
            ) + recv_buf[dirn, 0][:, col:col + nq].astype(jnp.float32)
            out_stage[q % 2] = jnp.maximum(val, 0.0)
            pltpu.make_async_copy(
                out_stage.at[q % 2], out_hbm.at[:, pl.ds(q * nq, nq)],
                out_sems.at[q % 2],
            ).start()

        for q in (2, 3):
            pltpu.make_async_copy(
                out_stage.at[q % 2], out_hbm.at[:, pl.ds(q * nq, nq)],
                out_sems.at[q % 2],
            ).wait()
        for dirn in (0, 1):
            rdma(dirn, 5).wait_send()
            rdma(dirn, 6).wait_send()

    return pl.pallas_call(
        body,
        out_shape=jax.ShapeDtypeStruct((m_per, n), jnp.float32),
        in_specs=[
            pl.BlockSpec(memory_space=pl.ANY),
            pl.BlockSpec(memory_space=pltpu.VMEM),
        ],
        out_specs=pl.BlockSpec(memory_space=pl.ANY),
        scratch_shapes=[
            pltpu.VMEM((2, 2, m_per, k_per), jnp.bfloat16),
            pltpu.VMEM((2, 2, m_per, nh), jnp.bfloat16),
            pltpu.VMEM((2, 2, m_per, nh), jnp.bfloat16),
            pltpu.VMEM((2, m_per, nq), jnp.float32),
            pltpu.SemaphoreType.DMA((2, 2)),
            pltpu.SemaphoreType.DMA((2, 2)),
            pltpu.SemaphoreType.DMA((2, 2)),
            pltpu.SemaphoreType.DMA((2,)),
            pltpu.SemaphoreType.REGULAR,
            pltpu.SemaphoreType.REGULAR,
        ],
        compiler_params=pltpu.CompilerParams(
            collective_id=0,
            vmem_limit_bytes=64 * 1024 * 1024,
        ),
    )(x, w_mat)


# device time: 379593 ns/iter; 1.0549x vs baseline; 1.0549x over previous
import jax
import jax.numpy as jnp
from jax import lax
from jax.experimental import pallas as pl
from jax.experimental.pallas import tpu as pltpu

N_DEV = 8


def kernel(x, w_mat):
    x = x.astype(jnp.bfloat16)
    w_mat = w_mat.astype(jnp.bfloat16)
    m, k_per = x.shape
    n = w_mat.shape[1]
    m_per = m // N_DEV
    nh = n // 2
    nq = n // 4

    def body(x_hbm, w_ref, out_hbm, xt, send_buf, recv_buf, out_stage,
             xt_sems, send_sems, recv_sems, out_sems, credit_r, credit_l):
        d = lax.axis_index("i")
        left = lax.rem(d + N_DEV - 1, N_DEV)
        right = lax.rem(d + 1, N_DEV)

        dst = (right, left)
        upstream = (left, right)
        credits = (credit_r, credit_l)

        def chunk(dirn, s):
            return lax.rem(d + 7 - s, N_DEV) if dirn == 0 else \
                   lax.rem(d + 1 + s, N_DEV)

        def xt_copy(dirn, s):
            return pltpu.make_async_copy(
                x_hbm.at[pl.ds(chunk(dirn, s) * m_per, m_per), :],
                xt.at[dirn, s % 2],
                xt_sems.at[dirn, s % 2],
            )

        def rdma(dirn, s, sub):
            return pltpu.make_async_remote_copy(
                src_ref=send_buf.at[dirn, s % 2, :, pl.ds(sub * nq, nq)],
                dst_ref=recv_buf.at[dirn, s % 2, :, pl.ds(sub * nq, nq)],
                send_sem=send_sems.at[dirn, s % 2, sub],
                recv_sem=recv_sems.at[dirn, s % 2, sub],
                device_id=(dst[dirn],),
                device_id_type=pl.DeviceIdType.MESH,
            )

        for s in (0, 1):
            for dirn in (0, 1):
                xt_copy(dirn, s).start()

        for dirn in (0, 1):
            xt_copy(dirn, 0).wait()
            send_buf[dirn, 0] = jnp.dot(
                xt[dirn, 0],
                w_ref[:, dirn * nh:(dirn + 1) * nh],
                preferred_element_type=jnp.float32,
            ).astype(jnp.bfloat16)

        barrier_sem = pltpu.get_barrier_semaphore()
        for nbr in (left, right):
            pl.semaphore_signal(
                barrier_sem, inc=1,
                device_id=(nbr,), device_id_type=pl.DeviceIdType.MESH,
            )
        pl.semaphore_wait(barrier_sem, 2)

        sub_order = ((0, 0), (1, 0), (0, 1), (1, 1))

        for s in range(7):
            if s > 0:
                for dirn in (0, 1):
                    if s >= 2:
                        for sub in (0, 1):
                            rdma(dirn, s - 2, sub).wait_send()
                    xt_copy(dirn, s).wait()
                    send_buf[dirn, s % 2] = jnp.dot(
                        xt[dirn, s % 2],
                        w_ref[:, dirn * nh:(dirn + 1) * nh],
                        preferred_element_type=jnp.float32,
                    ).astype(jnp.bfloat16)
            if s + 2 <= 7:
                for dirn in (0, 1):
                    xt_copy(dirn, s + 2).start()
            for dirn, sub in sub_order:
                col = sub * nq
                if s > 0:
                    rdma(dirn, s - 1, sub).wait_recv()
                    send_buf[dirn, s % 2, :, col:col + nq] = (
                        send_buf[dirn, s % 2][:, col:col + nq]
                        + recv_buf[dirn, (s - 1) % 2][:, col:col + nq]
                    )
                    if s - 1 <= 4:
                        pl.semaphore_signal(
                            credits[dirn], inc=1,
                            device_id=(upstream[dirn],),
                            device_id_type=pl.DeviceIdType.MESH,
                        )
                if s >= 2:
                    pl.semaphore_wait(credits[dirn], 1)
                rdma(dirn, s, sub).start()
            if s == 6:
                for dirn in (0, 1):
                    for sub in (0, 1):
                        rdma(dirn, 5, sub).wait_send()
                    xt_copy(dirn, 7).wait()
                    send_buf[dirn, 1] = jnp.dot(
                        xt[dirn, 1],
                        w_ref[:, dirn * nh:(dirn + 1) * nh],
                        preferred_element_type=jnp.float32,
                    ).astype(jnp.bfloat16)

        def out_copy(q, slot):
            return pltpu.make_async_copy(
                out_stage.at[slot], out_hbm.at[:, pl.ds(q * nq, nq)],
                out_sems.at[slot],
            )

        for idx, (dirn, sub) in enumerate(sub_order):
            q = dirn * 2 + sub
            col = sub * nq
            slot = idx % 2
            if idx >= 2:
                prev_dirn, prev_sub = sub_order[idx - 2]
                out_copy(prev_dirn * 2 + prev_sub, slot).wait()
            rdma(dirn, 6, sub).wait_recv()
            val = (
                send_buf[dirn, 1][:, col:col + nq].astype(jnp.float32)
                + recv_buf[dirn, 0][:, col:col + nq].astype(jnp.float32)
            )
            out_stage[slot] = jnp.maximum(val, 0.0)
            out_copy(q, slot).start()

        for idx in (2, 3):
            dirn, sub = sub_order[idx]
            out_copy(dirn * 2 + sub, idx % 2).wait()
        for dirn in (0, 1):
            for sub in (0, 1):
                rdma(dirn, 6, sub).wait_send()

    return pl.pallas_call(
        body,
        out_shape=jax.ShapeDtypeStruct((m_per, n), jnp.float32),
        in_specs=[
            pl.BlockSpec(memory_space=pl.ANY),
            pl.BlockSpec(memory_space=pltpu.VMEM),
        ],
        out_specs=pl.BlockSpec(memory_space=pl.ANY),
        scratch_shapes=[
            pltpu.VMEM((2, 2, m_per, k_per), jnp.bfloat16),
            pltpu.VMEM((2, 2, m_per, nh), jnp.bfloat16),
            pltpu.VMEM((2, 2, m_per, nh), jnp.bfloat16),
            pltpu.VMEM((2, m_per, nq), jnp.float32),
            pltpu.SemaphoreType.DMA((2, 2)),
            pltpu.SemaphoreType.DMA((2, 2, 2)),
            pltpu.SemaphoreType.DMA((2, 2, 2)),
            pltpu.SemaphoreType.DMA((2,)),
            pltpu.SemaphoreType.REGULAR,
            pltpu.SemaphoreType.REGULAR,
        ],
        compiler_params=pltpu.CompilerParams(
            collective_id=0,
            vmem_limit_bytes=64 * 1024 * 1024,
        ),
    )(x, w_mat)


# device time: 364576 ns/iter; 1.0983x vs baseline; 1.0412x over previous
import jax
import jax.numpy as jnp
from jax import lax
from jax.experimental import pallas as pl
from jax.experimental.pallas import tpu as pltpu

N_DEV = 8


def kernel(x, w_mat):
    w_mat = w_mat.astype(jnp.bfloat16)
    m, k_per = x.shape
    n = w_mat.shape[1]
    m_per = m // N_DEV
    nh = n // 2
    nq = n // 4

    def body(x_hbm, w_ref, out_hbm, xt, xf32, send_buf, recv_buf, out_stage,
             xf_sem, send_sems, recv_sems, out_sems, credit_r, credit_l):
        d = lax.axis_index("i")
        left = lax.rem(d + N_DEV - 1, N_DEV)
        right = lax.rem(d + 1, N_DEV)

        dst = (right, left)
        upstream = (left, right)
        credits = (credit_r, credit_l)

        def chunk(dirn, s):
            return lax.rem(d + 7 - s, N_DEV) if dirn == 0 else \
                   lax.rem(d + 1 + s, N_DEV)

        def fetch_tile(dirn, s):
            cp = pltpu.make_async_copy(
                x_hbm.at[pl.ds(chunk(dirn, s) * m_per, m_per), :],
                xf32, xf_sem,
            )
            cp.start()
            cp.wait()
            xt[dirn] = xf32[...].astype(jnp.bfloat16)

        def rdma(dirn, s, sub):
            return pltpu.make_async_remote_copy(
                src_ref=send_buf.at[dirn, s % 2, :, pl.ds(sub * nq, nq)],
                dst_ref=recv_buf.at[dirn, s % 2, :, pl.ds(sub * nq, nq)],
                send_sem=send_sems.at[dirn, s % 2, sub],
                recv_sem=recv_sems.at[dirn, s % 2, sub],
                device_id=(dst[dirn],),
                device_id_type=pl.DeviceIdType.MESH,
            )

        for dirn in (0, 1):
            fetch_tile(dirn, 0)
            send_buf[dirn, 0] = jnp.dot(
                xt[dirn],
                w_ref[:, dirn * nh:(dirn + 1) * nh],
                preferred_element_type=jnp.float32,
            ).astype(jnp.bfloat16)

        barrier_sem = pltpu.get_barrier_semaphore()
        for nbr in (left, right):
            pl.semaphore_signal(
                barrier_sem, inc=1,
                device_id=(nbr,), device_id_type=pl.DeviceIdType.MESH,
            )
        pl.semaphore_wait(barrier_sem, 2)

        sub_order = ((0, 0), (1, 0), (0, 1), (1, 1))

        for s in range(7):
            if s > 0:
                for dirn in (0, 1):
                    if s >= 2:
                        for sub in (0, 1):
                            rdma(dirn, s - 2, sub).wait_send()
                    fetch_tile(dirn, s)
                    send_buf[dirn, s % 2] = jnp.dot(
                        xt[dirn],
                        w_ref[:, dirn * nh:(dirn + 1) * nh],
                        preferred_element_type=jnp.float32,
                    ).astype(jnp.bfloat16)
            for dirn, sub in sub_order:
                col = sub * nq
                if s > 0:
                    rdma(dirn, s - 1, sub).wait_recv()
                    send_buf[dirn, s % 2, :, col:col + nq] = (
                        send_buf[dirn, s % 2][:, col:col + nq]
                        + recv_buf[dirn, (s - 1) % 2][:, col:col + nq]
                    )
                    if s - 1 <= 4:
                        pl.semaphore_signal(
                            credits[dirn], inc=1,
                            device_id=(upstream[dirn],),
                            device_id_type=pl.DeviceIdType.MESH,
                        )
                if s >= 2:
                    pl.semaphore_wait(credits[dirn], 1)
                rdma(dirn, s, sub).start()
            if s == 6:
                for dirn in (0, 1):
                    for sub in (0, 1):
                        rdma(dirn, 5, sub).wait_send()
                    fetch_tile(dirn, 7)
                    send_buf[dirn, 1] = jnp.dot(
                        xt[dirn],
                        w_ref[:, dirn * nh:(dirn + 1) * nh],
                        preferred_element_type=jnp.float32,
                    ).astype(jnp.bfloat16)

        def out_copy(q, slot):
            return pltpu.make_async_copy(
                out_stage.at[slot], out_hbm.at[:, pl.ds(q * nq, nq)],
                out_sems.at[slot],
            )

        for idx, (dirn, sub) in enumerate(sub_order):
            q = dirn * 2 + sub
            col = sub * nq
            slot = idx % 2
            if idx >= 2:
                prev_dirn, prev_sub = sub_order[idx - 2]
                out_copy(prev_dirn * 2 + prev_sub, slot).wait()
            rdma(dirn, 6, sub).wait_recv()
            val = (
                send_buf[dirn, 1][:, col:col + nq].astype(jnp.float32)
                + recv_buf[dirn, 0][:, col:col + nq].astype(jnp.float32)
            )
            out_stage[slot] = jnp.maximum(val, 0.0)
            out_copy(q, slot).start()

        for idx in (2, 3):
            dirn, sub = sub_order[idx]
            out_copy(dirn * 2 + sub, idx % 2).wait()
        for dirn in (0, 1):
            for sub in (0, 1):
                rdma(dirn, 6, sub).wait_send()

    return pl.pallas_call(
        body,
        out_shape=jax.ShapeDtypeStruct((m_per, n), jnp.float32),
        in_specs=[
            pl.BlockSpec(memory_space=pl.ANY),
            pl.BlockSpec(memory_space=pltpu.VMEM),
        ],
        out_specs=pl.BlockSpec(memory_space=pl.ANY),
        scratch_shapes=[
            pltpu.VMEM((2, m_per, k_per), jnp.bfloat16),
            pltpu.VMEM((m_per, k_per), jnp.float32),
            pltpu.VMEM((2, 2, m_per, nh), jnp.bfloat16),
            pltpu.VMEM((2, 2, m_per, nh), jnp.bfloat16),
            pltpu.VMEM((2, m_per, nq), jnp.float32),
            pltpu.SemaphoreType.DMA,
            pltpu.SemaphoreType.DMA((2, 2, 2)),
            pltpu.SemaphoreType.DMA((2, 2, 2)),
            pltpu.SemaphoreType.DMA((2,)),
            pltpu.SemaphoreType.REGULAR,
            pltpu.SemaphoreType.REGULAR,
        ],
        compiler_params=pltpu.CompilerParams(
            collective_id=0,
            vmem_limit_bytes=64 * 1024 * 1024,
        ),
    )(x, w_mat)


# device time: 349420 ns/iter; 1.1460x vs baseline; 1.0434x over previous
import jax
import jax.numpy as jnp
from jax import lax
from jax.experimental import pallas as pl
from jax.experimental.pallas import tpu as pltpu

N_DEV = 8


def kernel(x, w_mat):
    m, k_per = x.shape
    n = w_mat.shape[1]
    m_per = m // N_DEV
    nh = n // 2
    nq = n // 4

    def body(x_hbm, w_hbm, out_hbm, xt, xf32, w_ref, send_buf, recv_buf,
             out_stage, xf_sem, send_sems, recv_sems, out_sems,
             credit_r, credit_l):
        d = lax.axis_index("i")
        left = lax.rem(d + N_DEV - 1, N_DEV)
        right = lax.rem(d + 1, N_DEV)

        dst = (right, left)
        upstream = (left, right)
        credits = (credit_r, credit_l)

        def chunk(dirn, s):
            return lax.rem(d + 7 - s, N_DEV) if dirn == 0 else \
                   lax.rem(d + 1 + s, N_DEV)

        def fetch_tile(dirn, s):
            cp = pltpu.make_async_copy(
                x_hbm.at[pl.ds(chunk(dirn, s) * m_per, m_per), :],
                xf32, xf_sem,
            )
            cp.start()
            cp.wait()
            xt[dirn] = xf32[...].astype(jnp.bfloat16)

        def cast_w_strip(j):
            cp = pltpu.make_async_copy(
                w_hbm.at[:, pl.ds(j * nq, nq)], xf32, xf_sem,
            )
            cp.start()
            cp.wait()
            w_ref[:, j * nq:(j + 1) * nq] = xf32[...].astype(jnp.bfloat16)

        def rdma(dirn, s, sub):
            return pltpu.make_async_remote_copy(
                src_ref=send_buf.at[dirn, s % 2, :, pl.ds(sub * nq, nq)],
                dst_ref=recv_buf.at[dirn, s % 2, :, pl.ds(sub * nq, nq)],
                send_sem=send_sems.at[dirn, s % 2, sub],
                recv_sem=recv_sems.at[dirn, s % 2, sub],
                device_id=(dst[dirn],),
                device_id_type=pl.DeviceIdType.MESH,
            )

        barrier_sem = pltpu.get_barrier_semaphore()
        for nbr in (left, right):
            pl.semaphore_signal(
                barrier_sem, inc=1,
                device_id=(nbr,), device_id_type=pl.DeviceIdType.MESH,
            )
        pl.semaphore_wait(barrier_sem, 2)

        fetch_tile(0, 0)
        for dirn, sub in ((0, 0), (1, 0), (0, 1), (1, 1)):
            col = sub * nq
            cast_w_strip(dirn * 2 + sub)
            if dirn == 1 and sub == 0:
                fetch_tile(1, 0)
            send_buf[dirn, 0, :, col:col + nq] = jnp.dot(
                xt[dirn],
                w_ref[:, dirn * nh + col:dirn * nh + col + nq],
                preferred_element_type=jnp.float32,
            ).astype(jnp.bfloat16)
            rdma(dirn, 0, sub).start()

        sub_order = ((0, 0), (1, 0), (0, 1), (1, 1))

        for s in range(1, 7):
            for dirn in (0, 1):
                if s >= 2:
                    for sub in (0, 1):
                        rdma(dirn, s - 2, sub).wait_send()
                fetch_tile(dirn, s)
                send_buf[dirn, s % 2] = jnp.dot(
                    xt[dirn],
                    w_ref[:, dirn * nh:(dirn + 1) * nh],
                    preferred_element_type=jnp.float32,
                ).astype(jnp.bfloat16)
            for dirn, sub in sub_order:
                col = sub * nq
                rdma(dirn, s - 1, sub).wait_recv()
                send_buf[dirn, s % 2, :, col:col + nq] = (
                    send_buf[dirn, s % 2][:, col:col + nq]
                    + recv_buf[dirn, (s - 1) % 2][:, col:col + nq]
                )
                if s - 1 <= 4:
                    pl.semaphore_signal(
                        credits[dirn], inc=1,
                        device_id=(upstream[dirn],),
                        device_id_type=pl.DeviceIdType.MESH,
                    )
                if s >= 2:
                    pl.semaphore_wait(credits[dirn], 1)
                rdma(dirn, s, sub).start()
            if s == 6:
                for dirn in (0, 1):
                    for sub in (0, 1):
                        rdma(dirn, 5, sub).wait_send()
                    fetch_tile(dirn, 7)
                    send_buf[dirn, 1] = jnp.dot(
                        xt[dirn],
                        w_ref[:, dirn * nh:(dirn + 1) * nh],
                        preferred_element_type=jnp.float32,
                    ).astype(jnp.bfloat16)

        def out_copy(q, slot):
            return pltpu.make_async_copy(
                out_stage.at[slot], out_hbm.at[:, pl.ds(q * nq, nq)],
                out_sems.at[slot],
            )

        for idx, (dirn, sub) in enumerate(sub_order):
            q = dirn * 2 + sub
            col = sub * nq
            slot = idx % 2
            if idx >= 2:
                prev_dirn, prev_sub = sub_order[idx - 2]
                out_copy(prev_dirn * 2 + prev_sub, slot).wait()
            rdma(dirn, 6, sub).wait_recv()
            val = (
                send_buf[dirn, 1][:, col:col + nq].astype(jnp.float32)
                + recv_buf[dirn, 0][:, col:col + nq].astype(jnp.float32)
            )
            out_stage[slot] = jnp.maximum(val, 0.0)
            out_copy(q, slot).start()

        for idx in (2, 3):
            dirn, sub = sub_order[idx]
            out_copy(dirn * 2 + sub, idx % 2).wait()
        for dirn in (0, 1):
            for sub in (0, 1):
                rdma(dirn, 6, sub).wait_send()

    return pl.pallas_call(
        body,
        out_shape=jax.ShapeDtypeStruct((m_per, n), jnp.float32),
        in_specs=[
            pl.BlockSpec(memory_space=pl.ANY),
            pl.BlockSpec(memory_space=pl.ANY),
        ],
        out_specs=pl.BlockSpec(memory_space=pl.ANY),
        scratch_shapes=[
            pltpu.VMEM((2, m_per, k_per), jnp.bfloat16),
            pltpu.VMEM((m_per, k_per), jnp.float32),
            pltpu.VMEM((k_per, n), jnp.bfloat16),
            pltpu.VMEM((2, 2, m_per, nh), jnp.bfloat16),
            pltpu.VMEM((2, 2, m_per, nh), jnp.bfloat16),
            pltpu.VMEM((2, m_per, nq), jnp.float32),
            pltpu.SemaphoreType.DMA,
            pltpu.SemaphoreType.DMA((2, 2, 2)),
            pltpu.SemaphoreType.DMA((2, 2, 2)),
            pltpu.SemaphoreType.DMA((2,)),
            pltpu.SemaphoreType.REGULAR,
            pltpu.SemaphoreType.REGULAR,
        ],
        compiler_params=pltpu.CompilerParams(
            collective_id=0,
            vmem_limit_bytes=64 * 1024 * 1024,
        ),
    )(x, w_mat)
